# baseline (speedup 1.0000x reference)
import functools

import jax
import jax.numpy as jnp
from jax import lax
from jax.experimental import pallas as pl
from jax.experimental.pallas import tpu as pltpu
from jax.experimental.pallas import tpu_sc as plsc

H = 128
DEPTH = 3
MAX_NB = 6
LANES = 16


def _in_mm_body(x_ref, w_ref, bin_ref, msg_ref):
    y = jnp.dot(x_ref[...], w_ref[...], preferred_element_type=jnp.float32)
    bin_ref[...] = y
    msg_ref[...] = jnp.maximum(y, 0.0)


def _input_matmul(fbonds, W_i):
    n, k = fbonds.shape
    bm = 1280
    return pl.pallas_call(
        _in_mm_body,
        grid=(n // bm,),
        in_specs=[pl.BlockSpec((bm, k), lambda i: (i, 0)),
                  pl.BlockSpec((k, H), lambda i: (0, 0))],
        out_specs=[pl.BlockSpec((bm, H), lambda i: (i, 0)),
                   pl.BlockSpec((bm, H), lambda i: (i, 0))],
        out_shape=[jax.ShapeDtypeStruct((n, H), jnp.float32),
                   jax.ShapeDtypeStruct((n, H), jnp.float32)],
    )(fbonds, W_i)


def _upd_mm_body(nei_ref, w_ref, bin_ref, msg_ref):
    y = jnp.dot(nei_ref[...], w_ref[...], preferred_element_type=jnp.float32)
    msg_ref[...] = jnp.maximum(bin_ref[...] + y, 0.0)


def _update_matmul(nei, W_h, binput):
    n = binput.shape[0]
    bm = 1280
    return pl.pallas_call(
        _upd_mm_body,
        grid=(n // bm,),
        in_specs=[pl.BlockSpec((bm, H), lambda i: (i, 0)),
                  pl.BlockSpec((H, H), lambda i: (0, 0)),
                  pl.BlockSpec((bm, H), lambda i: (i, 0))],
        out_specs=pl.BlockSpec((bm, H), lambda i: (i, 0)),
        out_shape=jax.ShapeDtypeStruct((n, H), jnp.float32),
    )(nei, W_h, binput)


def _out_body(mols_per_blk, atoms_per_mol, f_ref, n_ref, w1_ref, w2_ref,
              b_ref, out_ref):
    h = jnp.dot(f_ref[...], w1_ref[...], preferred_element_type=jnp.float32)
    h = h + jnp.dot(n_ref[...], w2_ref[...], preferred_element_type=jnp.float32)
    h = jnp.maximum(h + b_ref[...], 0.0)
    bm = mols_per_blk * atoms_per_mol
    r = lax.broadcasted_iota(jnp.int32, (mols_per_blk, bm), 0)
    c = lax.broadcasted_iota(jnp.int32, (mols_per_blk, bm), 1)
    pool = jnp.where(c // atoms_per_mol == r, 1.0 / atoms_per_mol, 0.0)
    out_ref[...] = jnp.dot(pool.astype(jnp.float32), h,
                           preferred_element_type=jnp.float32)


def _output_pool(fatoms, nei, W_o1, W_o2, b_o, n_mols, atoms_per_mol):
    n, fd = fatoms.shape
    mols_per_blk = 80
    bm = mols_per_blk * atoms_per_mol
    body = functools.partial(_out_body, mols_per_blk, atoms_per_mol)
    return pl.pallas_call(
        body,
        grid=(n // bm,),
        in_specs=[pl.BlockSpec((bm, fd), lambda i: (i, 0)),
                  pl.BlockSpec((bm, H), lambda i: (i, 0)),
                  pl.BlockSpec((fd, H), lambda i: (0, 0)),
                  pl.BlockSpec((H, H), lambda i: (0, 0)),
                  pl.BlockSpec((1, H), lambda i: (0, 0))],
        out_specs=pl.BlockSpec((mols_per_blk, H), lambda i: (i, 0)),
        out_shape=jax.ShapeDtypeStruct((n_mols, H), jnp.float32),
    )(fatoms, nei, W_o1, W_o2, b_o)


def _make_gather_sum(n_out, chunk):
    info = plsc.get_sparse_core_info()
    nc, ns = info.num_cores, info.num_subcores
    nw = nc * ns
    per_w = n_out // nw
    n_chunks = per_w // chunk
    assert n_chunks % 2 == 0 and per_w % chunk == 0
    mesh = plsc.VectorSubcoreMesh(core_axis_name="c", subcore_axis_name="s")

    @functools.partial(
        pl.kernel, mesh=mesh,
        out_type=jax.ShapeDtypeStruct((n_out, H), jnp.float32),
        scratch_types=[
            pltpu.VMEM((MAX_NB * per_w,), jnp.int32),
            pltpu.VMEM((2, MAX_NB, chunk, H), jnp.float32),
            pltpu.VMEM((2, chunk, H), jnp.float32),
            pltpu.SemaphoreType.DMA,
            pltpu.SemaphoreType.DMA,
        ],
    )
    def gather_sum(table_hbm, idx_hbm, out_hbm, idx_v, rows_v, acc_v,
                   sem0, sem1):
        wid = lax.axis_index("s") * nc + lax.axis_index("c")
        base_w = wid * per_w
        sems = (sem0, sem1)

        for j in range(MAX_NB):
            pltpu.sync_copy(idx_hbm.at[pl.ds(j * n_out + base_w, per_w)],
                            idx_v.at[pl.ds(j * per_w, per_w)])

        def fire(c, b):
            for j in range(MAX_NB):
                pltpu.async_copy(
                    table_hbm.at[idx_v.at[pl.ds(j * per_w + c * chunk, chunk)]],
                    rows_v.at[b, j], sems[b])

        def drain(b):
            for j in range(MAX_NB):
                pltpu.make_async_copy(
                    table_hbm.at[idx_v.at[pl.ds(0, chunk)]],
                    rows_v.at[b, j], sems[b]).wait()

        def process(c, b):
            def bond_body(cb, carry2):
                for hh in range(H // LANES):
                    s = rows_v[b, 0, cb, pl.ds(hh * LANES, LANES)]
                    for j in range(1, MAX_NB):
                        s = s + rows_v[b, j, cb, pl.ds(hh * LANES, LANES)]
                    acc_v[b, cb, pl.ds(hh * LANES, LANES)] = s
                return carry2

            lax.fori_loop(0, chunk, bond_body, 0, unroll=2)
            pltpu.sync_copy(acc_v.at[b],
                            out_hbm.at[pl.ds(base_w + c * chunk, chunk)])

        fire(0, 0)

        def pair_body(p, carry):
            c0 = 2 * p
            fire(c0 + 1, 1)
            drain(0)
            process(c0, 0)

            @pl.when(c0 + 2 < n_chunks)
            def _():
                fire(c0 + 2, 0)

            drain(1)
            process(c0 + 1, 1)
            return carry

        lax.fori_loop(0, n_chunks // 2, pair_body, 0)

    return gather_sum


def kernel(fatoms, fbonds, agraph, bgraph, scope, W_i, W_h, W_o, b_o):
    n_atoms, fdim = fatoms.shape
    n_bonds = bgraph.shape[0]
    n_mols = scope.shape[0]
    atoms_per_mol = n_atoms // n_mols

    nb_pad = ((n_bonds + 2047) // 2048) * 2048
    bidx = jnp.pad(bgraph.T, ((0, 0), (0, nb_pad - n_bonds))).reshape(-1)
    na_pad = ((n_atoms + 2047) // 2048) * 2048
    aidx = jnp.pad(agraph.T, ((0, 0), (0, na_pad - n_atoms))).reshape(-1)

    binput, message = _input_matmul(fbonds, W_i)

    gs_bonds = _make_gather_sum(nb_pad, chunk=32)
    for _ in range(DEPTH - 1):
        nei = gs_bonds(message, bidx)
        message = _update_matmul(nei, W_h, binput)

    gs_atoms = _make_gather_sum(na_pad, chunk=32)
    nei_a = gs_atoms(message, aidx)

    return _output_pool(fatoms, nei_a, W_o[:fdim], W_o[fdim:],
                        b_o.reshape(1, H), n_mols, atoms_per_mol)


# speedup vs baseline: 1.0580x; 1.0580x over previous
import functools

import jax
import jax.numpy as jnp
from jax import lax
from jax.experimental import pallas as pl
from jax.experimental.pallas import tpu as pltpu
from jax.experimental.pallas import tpu_sc as plsc

H = 128
DEPTH = 3
MAX_NB = 6
LANES = 16


def _in_mm_body(x_ref, w_ref, bin_ref, msg_ref):
    y = jnp.dot(x_ref[...], w_ref[...], preferred_element_type=jnp.float32)
    bin_ref[...] = y
    msg_ref[...] = jnp.maximum(y, 0.0)


def _input_matmul(fbonds, W_i):
    n, k = fbonds.shape
    bm = 1280
    return pl.pallas_call(
        _in_mm_body,
        grid=(n // bm,),
        in_specs=[pl.BlockSpec((bm, k), lambda i: (i, 0)),
                  pl.BlockSpec((k, H), lambda i: (0, 0))],
        out_specs=[pl.BlockSpec((bm, H), lambda i: (i, 0)),
                   pl.BlockSpec((bm, H), lambda i: (i, 0))],
        out_shape=[jax.ShapeDtypeStruct((n, H), jnp.float32),
                   jax.ShapeDtypeStruct((n, H), jnp.float32)],
    )(fbonds, W_i)


def _upd_mm_body(nei_ref, w_ref, bin_ref, msg_ref):
    y = jnp.dot(nei_ref[...], w_ref[...], preferred_element_type=jnp.float32)
    msg_ref[...] = jnp.maximum(bin_ref[...] + y, 0.0)


def _update_matmul(nei, W_h, binput):
    n = binput.shape[0]
    bm = 1280
    return pl.pallas_call(
        _upd_mm_body,
        grid=(n // bm,),
        in_specs=[pl.BlockSpec((bm, H), lambda i: (i, 0)),
                  pl.BlockSpec((H, H), lambda i: (0, 0)),
                  pl.BlockSpec((bm, H), lambda i: (i, 0))],
        out_specs=pl.BlockSpec((bm, H), lambda i: (i, 0)),
        out_shape=jax.ShapeDtypeStruct((n, H), jnp.float32),
    )(nei, W_h, binput)


def _out_body(mols_per_blk, atoms_per_mol, f_ref, n_ref, w1_ref, w2_ref,
              b_ref, out_ref):
    h = jnp.dot(f_ref[...], w1_ref[...], preferred_element_type=jnp.float32)
    h = h + jnp.dot(n_ref[...], w2_ref[...], preferred_element_type=jnp.float32)
    h = jnp.maximum(h + b_ref[...], 0.0)
    bm = mols_per_blk * atoms_per_mol
    r = lax.broadcasted_iota(jnp.int32, (mols_per_blk, bm), 0)
    c = lax.broadcasted_iota(jnp.int32, (mols_per_blk, bm), 1)
    pool = jnp.where(c // atoms_per_mol == r, 1.0 / atoms_per_mol, 0.0)
    out_ref[...] = jnp.dot(pool.astype(jnp.float32), h,
                           preferred_element_type=jnp.float32)


def _output_pool(fatoms, nei, W_o1, W_o2, b_o, n_mols, atoms_per_mol):
    n, fd = fatoms.shape
    mols_per_blk = 80
    bm = mols_per_blk * atoms_per_mol
    body = functools.partial(_out_body, mols_per_blk, atoms_per_mol)
    return pl.pallas_call(
        body,
        grid=(n // bm,),
        in_specs=[pl.BlockSpec((bm, fd), lambda i: (i, 0)),
                  pl.BlockSpec((bm, H), lambda i: (i, 0)),
                  pl.BlockSpec((fd, H), lambda i: (0, 0)),
                  pl.BlockSpec((H, H), lambda i: (0, 0)),
                  pl.BlockSpec((1, H), lambda i: (0, 0))],
        out_specs=pl.BlockSpec((mols_per_blk, H), lambda i: (i, 0)),
        out_shape=jax.ShapeDtypeStruct((n_mols, H), jnp.float32),
    )(fatoms, nei, W_o1, W_o2, b_o)


def _make_gather_sum(n_out, chunk):
    info = plsc.get_sparse_core_info()
    nc, ns = info.num_cores, info.num_subcores
    nw = nc * ns
    per_w = n_out // nw
    n_chunks = per_w // chunk
    assert n_chunks % 2 == 0 and per_w % chunk == 0
    assert chunk * MAX_NB <= 128 and chunk % 8 == 0
    mesh = plsc.VectorSubcoreMesh(core_axis_name="c", subcore_axis_name="s")

    @functools.partial(
        pl.kernel, mesh=mesh,
        out_type=jax.ShapeDtypeStruct((n_out, H), jnp.float32),
        scratch_types=[
            pltpu.VMEM((MAX_NB * per_w,), jnp.int32),
            pltpu.VMEM((2, chunk * MAX_NB, H), jnp.float32),
            pltpu.VMEM((2, chunk, H), jnp.float32),
            pltpu.SemaphoreType.DMA,
            pltpu.SemaphoreType.DMA,
        ],
    )
    def gather_sum(table_hbm, idx_hbm, out_hbm, idx_v, rows_v, acc_v,
                   sem0, sem1):
        wid = lax.axis_index("s") * nc + lax.axis_index("c")
        base_w = wid * per_w
        sems = (sem0, sem1)
        cw = chunk * MAX_NB

        pltpu.sync_copy(idx_hbm.at[pl.ds(base_w * MAX_NB, per_w * MAX_NB)],
                        idx_v)

        def fire(c, b):
            pltpu.async_copy(table_hbm.at[idx_v.at[pl.ds(c * cw, cw)]],
                             rows_v.at[b], sems[b])

        def drain(b):
            pltpu.make_async_copy(table_hbm.at[idx_v.at[pl.ds(0, cw)]],
                                  rows_v.at[b], sems[b]).wait()

        def process(c, b):
            def bond_body(cb, carry2):
                for hh in range(H // LANES):
                    s = rows_v[b, cb * MAX_NB, pl.ds(hh * LANES, LANES)]
                    for j in range(1, MAX_NB):
                        s = s + rows_v[b, cb * MAX_NB + j,
                                       pl.ds(hh * LANES, LANES)]
                    acc_v[b, cb, pl.ds(hh * LANES, LANES)] = s
                return carry2

            lax.fori_loop(0, chunk, bond_body, 0, unroll=2)
            pltpu.sync_copy(acc_v.at[b],
                            out_hbm.at[pl.ds(base_w + c * chunk, chunk)])

        fire(0, 0)

        def pair_body(p, carry):
            c0 = 2 * p
            fire(c0 + 1, 1)
            drain(0)
            process(c0, 0)

            @pl.when(c0 + 2 < n_chunks)
            def _():
                fire(c0 + 2, 0)

            drain(1)
            process(c0 + 1, 1)
            return carry

        lax.fori_loop(0, n_chunks // 2, pair_body, 0)

    return gather_sum


def kernel(fatoms, fbonds, agraph, bgraph, scope, W_i, W_h, W_o, b_o):
    n_atoms, fdim = fatoms.shape
    n_bonds = bgraph.shape[0]
    n_mols = scope.shape[0]
    atoms_per_mol = n_atoms // n_mols

    nb_pad = ((n_bonds + 1023) // 1024) * 1024
    bidx = jnp.pad(bgraph.reshape(-1), (0, (nb_pad - n_bonds) * MAX_NB))
    na_pad = ((n_atoms + 1023) // 1024) * 1024
    aidx = jnp.pad(agraph.reshape(-1), (0, (na_pad - n_atoms) * MAX_NB))

    binput, message = _input_matmul(fbonds, W_i)

    gs_bonds = _make_gather_sum(nb_pad, chunk=16)
    for _ in range(DEPTH - 1):
        nei = gs_bonds(message, bidx)
        message = _update_matmul(nei, W_h, binput)

    gs_atoms = _make_gather_sum(na_pad, chunk=16)
    nei_a = gs_atoms(message, aidx)

    return _output_pool(fatoms, nei_a, W_o[:fdim], W_o[fdim:],
                        b_o.reshape(1, H), n_mols, atoms_per_mol)
